# Initial kernel scaffold; baseline (speedup 1.0000x reference)
import jax
import jax.numpy as jnp
from jax.experimental import pallas as pl


def kernel(logits, targets):
    raise NotImplementedError("write your pallas kernel here")


# speedup vs baseline: 1.2393x; 1.2393x over previous
import functools

import jax
import jax.numpy as jnp
from jax import lax
from jax.experimental import pallas as pl
from jax.experimental.pallas import tpu as pltpu

EOS = 0
PAD = 1000
EPS = 1e-08

BB = 8


def _body(l_ref, t_ref, lab_ref, eos_ref):
    step = pl.program_id(0)

    @pl.when(step == 0)
    def _init():
        lab_ref[...] = jnp.zeros((1, 1), jnp.float32)
        eos_ref[...] = jnp.zeros((1, 1), jnp.float32)

    bb, S, C = l_ref.shape
    l2 = l_ref[...].reshape(bb * S, C)
    m = jnp.max(l2, axis=1, keepdims=True)
    e = jnp.exp(l2 - m)
    Z = jnp.sum(e, axis=1, keepdims=True)
    e0 = e[:, 0:1]
    p0 = (e0 / Z).reshape(bb, S)
    s2 = ((jnp.sum(e * e, axis=1, keepdims=True) - e0 * e0) / (Z * Z)).reshape(bb, S)

    t = t_ref[...]
    hf = ((t != PAD) & (t != EOS)).astype(jnp.float32)

    e3 = e.reshape(bb, S, C)
    Z3 = Z.reshape(bb, S)

    lab_acc = 0.0
    for b in range(bb):
        tb = t[b:b + 1, :]
        ci = lax.broadcasted_iota(jnp.int32, (C, S), 0)
        oh = (ci == jnp.broadcast_to(tb, (C, S))).astype(jnp.float32)
        Ge = lax.dot_general(e3[b], oh, (((1,), (0,)), ((), ())),
                             preferred_element_type=jnp.float32)
        G = Ge / Z3[b][:, None]
        hfi = hf[b][:, None]
        hfj = hf[b][None, :]
        d = hfi * s2[b][:, None] + hfj - 2.0 * (hfi * hfj) * G
        lab_acc += jnp.mean(jnp.min(d, axis=1)) + jnp.mean(jnp.min(d, axis=0))

    logp = jnp.maximum(jnp.log(p0), -100.0)
    log1mp = jnp.maximum(jnp.log(1.0 - p0), -100.0)
    y = 1.0 - hf
    bce = -(y * logp + (1.0 - y) * log1mp)
    posf = (t == EOS).astype(jnp.float32)
    eos_b = (0.5 * jnp.sum(bce * posf, axis=1) / (jnp.sum(posf, axis=1) + EPS)
             + 0.5 * jnp.sum(bce * hf, axis=1) / (jnp.sum(hf, axis=1) + EPS))

    lab_ref[...] += jnp.reshape(lab_acc, (1, 1))
    eos_ref[...] += jnp.reshape(jnp.sum(eos_b), (1, 1))


_INTERPRET = False


def kernel(logits, targets):
    B, S, C = logits.shape
    grid = B // BB
    lab, eos = pl.pallas_call(
        _body,
        grid=(grid,),
        in_specs=[
            pl.BlockSpec((BB, S, C), lambda i: (i, 0, 0)),
            pl.BlockSpec((BB, S), lambda i: (i, 0)),
        ],
        out_specs=[
            pl.BlockSpec((1, 1), lambda i: (0, 0)),
            pl.BlockSpec((1, 1), lambda i: (0, 0)),
        ],
        out_shape=[
            jax.ShapeDtypeStruct((1, 1), jnp.float32),
            jax.ShapeDtypeStruct((1, 1), jnp.float32),
        ],
        interpret=_INTERPRET,
    )(logits, targets)
    return (lab[0, 0] / B, eos[0, 0] / B)
